# Initial kernel scaffold; baseline (speedup 1.0000x reference)
import jax
import jax.numpy as jnp
from jax.experimental import pallas as pl


def kernel(verts, edges, w0_w, w0_b, w1_w, w1_b):
    raise NotImplementedError("write your pallas kernel here")


# speedup vs baseline: 5.3941x; 5.3941x over previous
import functools

import jax
import jax.numpy as jnp
from jax import lax
from jax.experimental import pallas as pl
from jax.experimental.pallas import tpu as pltpu
from jax.experimental.pallas import tpu_sc as plsc

NC = 2
NS = 16
NW = NC * NS
CH = 128


def _proj_body(x_ref, w0t_ref, w1t_ref, b0_ref, b1_ref, o0_ref, o1_ref):
    x = x_ref[...]
    o0_ref[...] = jnp.dot(x, w0t_ref[...], preferred_element_type=jnp.float32) + b0_ref[...]
    o1_ref[...] = jnp.dot(x, w1t_ref[...], preferred_element_type=jnp.float32) + b1_ref[...]


def _combine_body(vw0_ref, p0_ref, p1_ref, o_ref):
    s = p0_ref[...] + p1_ref[...]
    o_ref[...] = vw0_ref[...] + jnp.concatenate([s, jnp.zeros_like(s)], axis=1)


def _make_agg(vpad, d_sup, per_w):
    n_chunks = per_w // CH
    rows_per_tile = vpad // NS
    mesh = plsc.VectorSubcoreMesh(core_axis_name="c", subcore_axis_name="s")

    @functools.partial(
        pl.kernel,
        mesh=mesh,
        out_type=jax.ShapeDtypeStruct((NC, vpad, d_sup), jnp.float32),
        scratch_types=[
            pltpu.VMEM((CH,), jnp.int32),
            pltpu.VMEM((CH,), jnp.int32),
            pltpu.VMEM((CH, d_sup), jnp.float32),
            pltpu.VMEM_SHARED((vpad, d_sup), jnp.float32),
            pltpu.SemaphoreType.DMA,
        ],
        compiler_params=pltpu.CompilerParams(use_tc_tiling_on_sc=False),
    )
    def agg(w1_hbm, dst_hbm, src_hbm, zeros_hbm, out_hbm, dsti, srci, rows, acc, sem):
        c = lax.axis_index("c")
        s = lax.axis_index("s")
        wid = s * NC + c
        base_w = wid * per_w

        r0 = s * rows_per_tile
        pltpu.sync_copy(zeros_hbm.at[pl.ds(r0, rows_per_tile)],
                        acc.at[pl.ds(r0, rows_per_tile)])
        plsc.subcore_barrier()

        def chunk(k, _):
            off = base_w + k * CH
            pltpu.sync_copy(dst_hbm.at[pl.ds(off, CH)], dsti)
            pltpu.sync_copy(src_hbm.at[pl.ds(off, CH)], srci)
            pltpu.async_copy(w1_hbm.at[srci], rows, sem).wait()
            pltpu.sync_copy(rows, acc.at[dsti], add=True)
            return _

        lax.fori_loop(0, n_chunks, chunk, None)
        plsc.subcore_barrier()
        pltpu.sync_copy(acc.at[pl.ds(r0, rows_per_tile)],
                        out_hbm.at[c, pl.ds(r0, rows_per_tile)])

    return agg


@jax.jit
def kernel(verts, edges, w0_w, w0_b, w1_w, w1_b):
    v, d_in = verts.shape
    d_out = w0_w.shape[0]
    d_sup = w1_w.shape[0]
    e = edges.shape[0]
    vb = 1000
    grid = v // vb

    vw0, vw1 = pl.pallas_call(
        _proj_body,
        grid=(grid,),
        in_specs=[
            pl.BlockSpec((vb, d_in), lambda i: (i, 0)),
            pl.BlockSpec((d_in, d_out), lambda i: (0, 0)),
            pl.BlockSpec((d_in, d_sup), lambda i: (0, 0)),
            pl.BlockSpec((1, d_out), lambda i: (0, 0)),
            pl.BlockSpec((1, d_sup), lambda i: (0, 0)),
        ],
        out_specs=[
            pl.BlockSpec((vb, d_out), lambda i: (i, 0)),
            pl.BlockSpec((vb, d_sup), lambda i: (i, 0)),
        ],
        out_shape=[
            jax.ShapeDtypeStruct((v, d_out), jnp.float32),
            jax.ShapeDtypeStruct((v, d_sup), jnp.float32),
        ],
    )(verts, w0_w.T, w1_w.T, w0_b[None, :], w1_b[None, :])

    dst = jnp.concatenate([edges[:, 0], edges[:, 1]])
    src = jnp.concatenate([edges[:, 1], edges[:, 0]])
    per_w = -(-2 * e // (NW * CH)) * CH
    total = per_w * NW
    pad_n = total - 2 * e
    vpad = -(-(v + 1) // (NS * 8)) * (NS * 8)
    dst = jnp.concatenate([dst, jnp.full((pad_n,), v, jnp.int32)])
    src = jnp.concatenate([src, jnp.zeros((pad_n,), jnp.int32)])
    zeros = jnp.zeros((vpad, d_sup), jnp.float32)

    partials = _make_agg(vpad, d_sup, per_w)(vw1, dst, src, zeros)

    out = pl.pallas_call(
        _combine_body,
        grid=(grid,),
        in_specs=[
            pl.BlockSpec((vb, d_out), lambda i: (i, 0)),
            pl.BlockSpec((vb, d_sup), lambda i: (i, 0)),
            pl.BlockSpec((vb, d_sup), lambda i: (i, 0)),
        ],
        out_specs=pl.BlockSpec((vb, d_out), lambda i: (i, 0)),
        out_shape=jax.ShapeDtypeStruct((v, d_out), jnp.float32),
    )(vw0, partials[0, :v], partials[1, :v])
    return out
